# Initial kernel scaffold; baseline (speedup 1.0000x reference)
import jax
import jax.numpy as jnp
from jax import lax
from jax.experimental import pallas as pl
from jax.experimental.pallas import tpu as pltpu
from jax.experimental.pallas import tpu_sc as plsc

_N_SP = 50000
_N_RX = 1600000
_NNZ = 6400000

_L = 16
_NW = 32

_RB = 2000
_CB = 2048
_SW = 128
_NSTR = _CB // _SW
_ACC_SLICE = 3136
_ACC_PAD = 16 * _ACC_SLICE

_mesh = plsc.VectorSubcoreMesh(core_axis_name="c", subcore_axis_name="s")


def _rates_body(coeffs_hbm, mults_hbm, abund_hbm, scal_hbm, rates_hbm,
                abund_v, scal_v):
    pltpu.sync_copy(abund_hbm, abund_v)
    pltpu.sync_copy(scal_hbm, scal_v)
    tm = scal_v[pl.ds(0, _L)]
    cr = scal_v[pl.ds(_L, _L)]
    ph = scal_v[pl.ds(2 * _L, _L)]

    def body(coeffs_v, mults_v, rates_v):
        @pl.loop(0, _RB, step=_L)
        def _(i):
            ofs = 3 * i + 3 * lax.iota(jnp.int32, _L)
            a = plsc.load_gather(coeffs_v, [ofs])
            b = plsc.load_gather(coeffs_v, [ofs + 1])
            g = plsc.load_gather(coeffs_v, [ofs + 2])
            i0 = plsc.load_gather(mults_v, [ofs])
            i1 = plsc.load_gather(mults_v, [ofs + 1])
            i2 = plsc.load_gather(mults_v, [ofs + 2])
            y0 = plsc.load_gather(abund_v, [i0])
            y1 = plsc.load_gather(abund_v, [i1])
            y2 = plsc.load_gather(abund_v, [i2])
            rate = a * jnp.exp(b * tm) + b * cr + g * ph
            rates_v[pl.ds(i, _L)] = rate * (y0 * y1 * y2)

    pltpu.emit_pipeline(
        body,
        grid=(_N_RX // _RB,),
        in_specs=[
            pl.BlockSpec((3 * _RB,), lambda i: (i,)),
            pl.BlockSpec((3 * _RB,), lambda i: (i,)),
        ],
        out_specs=[pl.BlockSpec((_RB,), lambda i: (i,))],
        core_axis_name=("c", "s"),
        dimension_semantics=(pltpu.PARALLEL,),
    )(coeffs_hbm, mults_hbm, rates_hbm)


def _matvec_body(rates_hbm, rows_hbm, cols_hbm, vals_hbm, out_hbm,
                 acc_sh, zero_v, gath_v):
    sid = lax.axis_index("s")
    cid = lax.axis_index("c")

    @pl.loop(0, _ACC_SLICE, step=_L)
    def _(k):
        zero_v[pl.ds(k, _L)] = jnp.zeros((_L,), jnp.float32)

    pltpu.sync_copy(zero_v, acc_sh.at[pl.ds(sid * _ACC_SLICE, _ACC_SLICE)])
    plsc.subcore_barrier()

    def body(rows_v, cols_v, vals_v):
        @pl.loop(0, _NSTR)
        def _(j):
            pltpu.sync_copy(
                rates_hbm.at[cols_v.at[pl.ds(j * _SW, _SW)]],
                gath_v.at[pl.ds(j * _SW, _SW)])

        @pl.loop(0, _CB, step=_L)
        def _(k):
            gath_v[pl.ds(k, _L)] = gath_v[pl.ds(k, _L)] * vals_v[pl.ds(k, _L)]

        @pl.loop(0, _NSTR)
        def _(j):
            pltpu.sync_copy(gath_v.at[pl.ds(j * _SW, _SW)],
                            acc_sh.at[rows_v.at[j]], add=True)

    pltpu.emit_pipeline(
        body,
        grid=(_NNZ // _CB,),
        in_specs=[
            pl.BlockSpec((_NSTR, _SW), lambda i: (i, 0)),
            pl.BlockSpec((_CB,), lambda i: (i,)),
            pl.BlockSpec((_CB,), lambda i: (i,)),
        ],
        out_specs=[],
        core_axis_name=("c", "s"),
        dimension_semantics=(pltpu.PARALLEL,),
    )(rows_hbm, cols_hbm, vals_hbm)

    plsc.subcore_barrier()
    pltpu.sync_copy(acc_sh.at[pl.ds(sid * _ACC_SLICE, _ACC_SLICE)],
                    out_hbm.at[cid, pl.ds(sid * _ACC_SLICE, _ACC_SLICE)])


def kernel(time, abundances, temperature, cr_rate, fuv_rate,
           visual_extinction, reactant_multipliers, rate_coeffs,
           inc_rows, inc_cols, inc_vals):
    del time
    coeffs_flat = rate_coeffs.reshape(-1)
    mults_flat = reactant_multipliers.astype(jnp.int32).reshape(-1)
    tm05 = jnp.broadcast_to(temperature - 0.5, (_L,))
    s_cr = jnp.broadcast_to(cr_rate, (_L,))
    s_ph = jnp.broadcast_to(fuv_rate * jnp.exp(-3.02 * visual_extinction),
                            (_L,))
    scal = jnp.concatenate([tm05, s_cr, s_ph]).astype(jnp.float32)

    rates = pl.kernel(
        _rates_body,
        out_type=jax.ShapeDtypeStruct((_N_RX,), jnp.float32),
        mesh=_mesh,
        scratch_types=[
            pltpu.VMEM((_N_SP,), jnp.float32),
            pltpu.VMEM((3 * _L,), jnp.float32),
        ],
    )(coeffs_flat, mults_flat, abundances, scal)

    rows2 = inc_rows.astype(jnp.int32).reshape(_NNZ // _SW, _SW)
    cols_flat = inc_cols.astype(jnp.int32)
    vals_flat = inc_vals

    partials = pl.kernel(
        _matvec_body,
        out_type=jax.ShapeDtypeStruct((2, _ACC_PAD), jnp.float32),
        mesh=_mesh,
        scratch_types=[
            pltpu.VMEM_SHARED((_ACC_PAD,), jnp.float32),
            pltpu.VMEM((_ACC_SLICE,), jnp.float32),
            pltpu.VMEM((_CB,), jnp.float32),
        ],
    )(rates, rows2, cols_flat, vals_flat)

    return partials[0, :_N_SP] + partials[1, :_N_SP]


# speedup vs baseline: 13.8044x; 13.8044x over previous
import dataclasses

import jax
import jax.numpy as jnp
from jax import lax
from jax.experimental import pallas as pl
from jax.experimental.pallas import tpu as pltpu
from jax.experimental.pallas import tpu_sc as plsc

_N_SP = 50000
_N_RX = 1600000
_NNZ = 6400000

_L = 16
_NW = 32

_RB = 2000
_CB = 2048
_SW = 128
_NSTR = _CB // _SW
_ACC_SLICE = 3136
_ACC_PAD = 16 * _ACC_SLICE

_mesh = plsc.VectorSubcoreMesh(core_axis_name="c", subcore_axis_name="s")

_cp = pltpu.CompilerParams()
if "needs_layout_passes" in pltpu.CompilerParams.__dataclass_fields__:
    _cp = dataclasses.replace(_cp, needs_layout_passes=False)


def _rates_body(coeffs_hbm, mults_hbm, abund_hbm, scal_hbm, rates_hbm,
                abund_v, scal_v):
    pltpu.sync_copy(abund_hbm, abund_v)
    pltpu.sync_copy(scal_hbm, scal_v)
    tm = scal_v[pl.ds(0, _L)]
    cr = scal_v[pl.ds(_L, _L)]
    ph = scal_v[pl.ds(2 * _L, _L)]

    def body(coeffs_v, mults_v, rates_v):
        @pl.loop(0, _RB, step=_L)
        def _(i):
            ofs = 3 * i + 3 * lax.iota(jnp.int32, _L)
            a = plsc.load_gather(coeffs_v, [ofs])
            b = plsc.load_gather(coeffs_v, [ofs + 1])
            g = plsc.load_gather(coeffs_v, [ofs + 2])
            i0 = plsc.load_gather(mults_v, [ofs])
            i1 = plsc.load_gather(mults_v, [ofs + 1])
            i2 = plsc.load_gather(mults_v, [ofs + 2])
            y0 = plsc.load_gather(abund_v, [i0])
            y1 = plsc.load_gather(abund_v, [i1])
            y2 = plsc.load_gather(abund_v, [i2])
            rate = a * jnp.exp(b * tm) + b * cr + g * ph
            rates_v[pl.ds(i, _L)] = rate * (y0 * y1 * y2)

    pltpu.emit_pipeline(
        body,
        grid=(_N_RX // _RB,),
        in_specs=[
            pl.BlockSpec((3 * _RB,), lambda i: (i,)),
            pl.BlockSpec((3 * _RB,), lambda i: (i,)),
        ],
        out_specs=[pl.BlockSpec((_RB,), lambda i: (i,))],
        core_axis_name=("c", "s"),
        dimension_semantics=(pltpu.PARALLEL,),
    )(coeffs_hbm, mults_hbm, rates_hbm)


def _matvec_body(rates_hbm, rows_hbm, cols_hbm, vals_hbm, out_hbm,
                 acc_sh, zero_v, gath_v):
    sid = lax.axis_index("s")
    cid = lax.axis_index("c")

    @pl.loop(0, _ACC_SLICE, step=_L)
    def _(k):
        zero_v[pl.ds(k, _L)] = jnp.zeros((_L,), jnp.float32)

    pltpu.sync_copy(zero_v, acc_sh.at[pl.ds(sid * _ACC_SLICE, _ACC_SLICE)])
    plsc.subcore_barrier()

    def body(rows_v, cols_v, vals_v):
        @pl.loop(0, _NSTR)
        def _(j):
            pltpu.sync_copy(
                rates_hbm.at[cols_v.at[pl.ds(j * _SW, _SW)]],
                gath_v.at[pl.ds(j * _SW, _SW)])

        @pl.loop(0, _CB, step=_L)
        def _(k):
            gath_v[pl.ds(k, _L)] = gath_v[pl.ds(k, _L)] * vals_v[pl.ds(k, _L)]

        @pl.loop(0, _NSTR)
        def _(j):
            pltpu.sync_copy(gath_v.at[pl.ds(j * _SW, _SW)],
                            acc_sh.at[rows_v.at[j]], add=True)

    pltpu.emit_pipeline(
        body,
        grid=(_NNZ // _CB,),
        in_specs=[
            pl.BlockSpec((_NSTR, _SW), lambda i: (i, 0)),
            pl.BlockSpec((_CB,), lambda i: (i,)),
            pl.BlockSpec((_CB,), lambda i: (i,)),
        ],
        out_specs=[],
        core_axis_name=("c", "s"),
        dimension_semantics=(pltpu.PARALLEL,),
    )(rows_hbm, cols_hbm, vals_hbm)

    plsc.subcore_barrier()
    pltpu.sync_copy(acc_sh.at[pl.ds(sid * _ACC_SLICE, _ACC_SLICE)], zero_v)
    pltpu.sync_copy(
        zero_v,
        out_hbm.at[pl.ds(cid * _ACC_PAD + sid * _ACC_SLICE, _ACC_SLICE)])


def kernel(time, abundances, temperature, cr_rate, fuv_rate,
           visual_extinction, reactant_multipliers, rate_coeffs,
           inc_rows, inc_cols, inc_vals):
    del time
    coeffs_flat = rate_coeffs.reshape(-1)
    mults_flat = reactant_multipliers.astype(jnp.int32).reshape(-1)
    tm05 = jnp.broadcast_to(temperature - 0.5, (_L,))
    s_cr = jnp.broadcast_to(cr_rate, (_L,))
    s_ph = jnp.broadcast_to(fuv_rate * jnp.exp(-3.02 * visual_extinction),
                            (_L,))
    scal = jnp.concatenate([tm05, s_cr, s_ph]).astype(jnp.float32)

    rates = pl.kernel(
        _rates_body,
        out_type=jax.ShapeDtypeStruct((_N_RX,), jnp.float32),
        mesh=_mesh,
        scratch_types=[
            pltpu.VMEM((_N_SP,), jnp.float32),
            pltpu.VMEM((3 * _L,), jnp.float32),
        ],
        compiler_params=_cp,
    )(coeffs_flat, mults_flat, abundances, scal)

    rows2 = inc_rows.astype(jnp.int32).reshape(_NNZ // _SW, _SW)
    cols_flat = inc_cols.astype(jnp.int32)
    vals_flat = inc_vals

    partials = pl.kernel(
        _matvec_body,
        out_type=jax.ShapeDtypeStruct((2 * _ACC_PAD,), jnp.float32),
        mesh=_mesh,
        scratch_types=[
            pltpu.VMEM_SHARED((_ACC_PAD,), jnp.float32),
            pltpu.VMEM((_ACC_SLICE,), jnp.float32),
            pltpu.VMEM((_CB,), jnp.float32),
        ],
        compiler_params=_cp,
    )(rates, rows2, cols_flat, vals_flat)

    return partials[:_N_SP] + partials[_ACC_PAD:_ACC_PAD + _N_SP]
